# baseline (speedup 1.0000x reference)
import functools

import jax
import jax.numpy as jnp
from jax.experimental import pallas as pl
from jax.experimental.pallas import tpu as pltpu

_PAR2 = pltpu.CompilerParams(dimension_semantics=("parallel", "parallel"))
_PAR1 = pltpu.CompilerParams(dimension_semantics=("parallel",))

K = 16
_NEG_SLOPE = 0.2


def _leaky(v):
    return jnp.where(v >= 0, v, _NEG_SLOPE * v)


def _conv_body(posr_ref, pt_ref, ssqr_ref, a_ref, wst_ref, wet_ref,
               *refs, C, R, N, emit_mask, use_mask, k):
    if use_mask:
        wm_in_ref, out_ref = refs
    elif emit_mask:
        out_ref, wm_out_ref = refs
    else:
        (out_ref,) = refs
    i = pl.program_id(1)

    a_rows = a_ref[0, pl.ds(i * R, R), :]
    x_r = a_rows[:, :C]
    pos3_r = a_rows[:, C:C + 3]
    psq_r = a_rows[:, C + 3:C + 4]

    if use_mask:
        w = wm_in_ref[0]
    else:
        pos_r = posr_ref[0]
        ptm = pt_ref[0]
        pb = pos_r.astype(jnp.bfloat16).astype(jnp.float32)
        tb = ptm.astype(jnp.bfloat16).astype(jnp.float32)
        e = ((pb[:, 0:1] * tb[0:1, :] + pb[:, 1:2] * tb[1:2, :])
             + pb[:, 2:3] * tb[2:3, :])
        d = (psq_r + ssqr_ref[0]) - 2.0 * e
        d = jnp.maximum(d, 0.0)
        iota = jax.lax.broadcasted_iota(jnp.int32, (R, N), 1)
        mask = jnp.zeros((R, N), jnp.float32)
        for _ in range(k):
            dm = jnp.min(d, axis=1, keepdims=True)
            sel = jnp.min(jnp.where(d == dm, iota, N), axis=1, keepdims=True)
            hit = iota == sel
            mask = jnp.maximum(mask, hit.astype(jnp.float32))
            d = jnp.where(hit, jnp.inf, d)
        w = mask * jnp.float32(1.0 / k)
        if emit_mask:
            wm_out_ref[0] = w

    agg = jnp.dot(w, a_ref[0], preferred_element_type=jnp.float32)
    agg_x = agg[:, :C]
    agg_p = agg[:, C:C + 3]
    agg_sq = agg[:, C + 3:C + 4]
    rel_mean = pos3_r - agg_p
    meandsq = psq_r - 2.0 * jnp.sum(pos3_r * agg_p, axis=1,
                                    keepdims=True) + agg_sq
    g = jnp.concatenate([agg_x, rel_mean, meandsq], axis=1)
    val = (jnp.dot(x_r, wst_ref[...], preferred_element_type=jnp.float32)
           + jnp.dot(g, wet_ref[...], preferred_element_type=jnp.float32))
    out_ref[0] = _leaky(val)


def _edge_conv(aug, pos_pad, pos_t, ssq_row, w_self, w_edge, *,
               wmask=None, emit_mask=False, R=256, k=K):
    B, N, CA = aug.shape
    C = CA - 4
    Cout = w_self.shape[0]
    R = min(R, N)
    grid = (B, N // R)
    body = functools.partial(_conv_body, C=C, R=R, N=N,
                             emit_mask=emit_mask, use_mask=wmask is not None,
                             k=k)
    in_specs = [
        pl.BlockSpec((1, R, 8), lambda b, i: (b, i, 0)),
        pl.BlockSpec((1, 8, N), lambda b, i: (b, 0, 0)),
        pl.BlockSpec((1, 1, N), lambda b, i: (b, 0, 0)),
        pl.BlockSpec((1, N, CA), lambda b, i: (b, 0, 0)),
        pl.BlockSpec((C, Cout), lambda b, i: (0, 0)),
        pl.BlockSpec((CA, Cout), lambda b, i: (0, 0)),
    ]
    args = [pos_pad, pos_t, ssq_row, aug, w_self.T, w_edge.T]
    out_specs = [pl.BlockSpec((1, R, Cout), lambda b, i: (b, i, 0))]
    out_shapes = [jax.ShapeDtypeStruct((B, N, Cout), jnp.float32)]
    if wmask is not None:
        in_specs.append(pl.BlockSpec((1, R, N), lambda b, i: (b, i, 0)))
        args.append(wmask)
    elif emit_mask:
        out_specs.append(pl.BlockSpec((1, R, N), lambda b, i: (b, i, 0)))
        out_shapes.append(jax.ShapeDtypeStruct((B, N, N), jnp.float32))
    res = pl.pallas_call(
        body,
        grid=grid,
        in_specs=in_specs,
        out_specs=out_specs if len(out_specs) > 1 else out_specs[0],
        out_shape=out_shapes if len(out_shapes) > 1 else out_shapes[0],
        compiler_params=_PAR2,
    )(*args)
    if emit_mask and wmask is None:
        return res[0], res[1]
    return (res[0] if isinstance(res, (list, tuple)) else res), None


def _fps_body(pos_ref, idx1_ref, sel1_ref, idx2_ref, sel2_ref, *, N, N1, N2):
    xs = pos_ref[0, 0]
    ys = pos_ref[0, 1]
    zs = pos_ref[0, 2]

    def lin_of(n):
        m = n // 8
        return (jax.lax.broadcasted_iota(jnp.int32, (8, m), 0) * m
                + jax.lax.broadcasted_iota(jnp.int32, (8, m), 1))

    def run(xc, yc, zc, npoint, n):
        lin = lin_of(n)
        lin_sel = lin_of(npoint)

        def body(t, state):
            dists, idxacc, sx, sy, sz, lx, ly, lz = state
            dx = xc - lx
            dy = yc - ly
            dz = zc - lz
            d = (dx * dx + dy * dy) + dz * dz
            dists = jnp.minimum(dists, d)
            maxv = jnp.max(dists, axis=(0, 1), keepdims=True)
            nxt = jnp.min(jnp.where(dists == maxv, lin, n), axis=(0, 1),
                          keepdims=True)
            hit = lin == nxt
            lx = jnp.sum(jnp.where(hit, xc, 0.0), axis=(0, 1), keepdims=True)
            ly = jnp.sum(jnp.where(hit, yc, 0.0), axis=(0, 1), keepdims=True)
            lz = jnp.sum(jnp.where(hit, zc, 0.0), axis=(0, 1), keepdims=True)
            at = lin_sel == t
            idxacc = jnp.where(at, nxt, idxacc)
            sx = jnp.where(at, lx, sx)
            sy = jnp.where(at, ly, sy)
            sz = jnp.where(at, lz, sz)
            return dists, idxacc, sx, sy, sz, lx, ly, lz

        lx0 = xc[0:1, 0:1]
        ly0 = yc[0:1, 0:1]
        lz0 = zc[0:1, 0:1]
        m_sel = npoint // 8
        at0 = lin_sel == 0
        z_i = jnp.zeros((8, m_sel), jnp.int32)
        sx0 = jnp.where(at0, lx0, jnp.zeros((8, m_sel), jnp.float32))
        sy0 = jnp.where(at0, ly0, jnp.zeros((8, m_sel), jnp.float32))
        sz0 = jnp.where(at0, lz0, jnp.zeros((8, m_sel), jnp.float32))
        state = (jnp.full((8, n // 8), 1e10, jnp.float32), z_i,
                 sx0, sy0, sz0, lx0, ly0, lz0)
        state = jax.lax.fori_loop(1, npoint, body, state)
        return state[1], state[2], state[3], state[4]

    idx1, s1x, s1y, s1z = run(xs, ys, zs, N1, N)
    idx1_ref[0] = idx1
    sel1_ref[0, 0] = s1x
    sel1_ref[0, 1] = s1y
    sel1_ref[0, 2] = s1z
    idx2, s2x, s2y, s2z = run(s1x, s1y, s1z, N2, N1)
    idx2_ref[0] = idx2
    sel2_ref[0, 0] = s2x
    sel2_ref[0, 1] = s2y
    sel2_ref[0, 2] = s2z


def _fps_chain(pos_t):
    B, _, N = pos_t.shape
    N1, N2 = N // 2, N // 4
    pos4 = pos_t.reshape(B, 3, 8, N // 8)
    idx1, sel1, idx2, sel2 = pl.pallas_call(
        functools.partial(_fps_body, N=N, N1=N1, N2=N2),
        grid=(B,),
        in_specs=[pl.BlockSpec((1, 3, 8, N // 8), lambda b: (b, 0, 0, 0))],
        out_specs=(
            pl.BlockSpec((1, 8, N1 // 8), lambda b: (b, 0, 0)),
            pl.BlockSpec((1, 3, 8, N1 // 8), lambda b: (b, 0, 0, 0)),
            pl.BlockSpec((1, 8, N2 // 8), lambda b: (b, 0, 0)),
            pl.BlockSpec((1, 3, 8, N2 // 8), lambda b: (b, 0, 0, 0)),
        ),
        out_shape=(
            jax.ShapeDtypeStruct((B, 8, N1 // 8), jnp.int32),
            jax.ShapeDtypeStruct((B, 3, 8, N1 // 8), jnp.float32),
            jax.ShapeDtypeStruct((B, 8, N2 // 8), jnp.int32),
            jax.ShapeDtypeStruct((B, 3, 8, N2 // 8), jnp.float32),
        ),
        compiler_params=_PAR1,
    )(pos4)
    return (idx1.reshape(B, N1), sel1.reshape(B, 3, N1),
            idx2.reshape(B, N2), sel2.reshape(B, 3, N2))


def _up_body(posr_ref, pt_ref, ssqr_ref, fk_ref, fs_ref, wat_ref, wbt_ref,
             b_ref, out_ref, *, R, NK):
    pos_r = posr_ref[0]
    ptm = pt_ref[0]
    psq_r = ((pos_r[:, 0:1] * pos_r[:, 0:1] + pos_r[:, 1:2] * pos_r[:, 1:2])
             + pos_r[:, 2:3] * pos_r[:, 2:3])
    pb = pos_r.astype(jnp.bfloat16).astype(jnp.float32)
    tb = ptm.astype(jnp.bfloat16).astype(jnp.float32)
    e = ((pb[:, 0:1] * tb[0:1, :] + pb[:, 1:2] * tb[1:2, :])
         + pb[:, 2:3] * tb[2:3, :])
    d = (psq_r + ssqr_ref[0]) - 2.0 * e
    d = jnp.maximum(d, 0.0)
    dd = d
    iota = jax.lax.broadcasted_iota(jnp.int32, (R, NK), 1)
    mask = jnp.zeros((R, NK), jnp.float32)
    for _ in range(3):
        dm = jnp.min(dd, axis=1, keepdims=True)
        sel = jnp.min(jnp.where(dd == dm, iota, NK), axis=1, keepdims=True)
        hit = iota == sel
        mask = jnp.maximum(mask, hit.astype(jnp.float32))
        dd = jnp.where(hit, jnp.inf, dd)
    wfull = mask / (d + 1e-8)
    w3 = wfull / jnp.sum(wfull, axis=1, keepdims=True)
    interp = jnp.dot(w3, fk_ref[0], preferred_element_type=jnp.float32)
    val = (jnp.dot(interp, wat_ref[...], preferred_element_type=jnp.float32)
           + jnp.dot(fs_ref[0], wbt_ref[...],
                     preferred_element_type=jnp.float32)
           + b_ref[...])
    out_ref[0] = _leaky(val)


def _upsample(posu_pad, posk_t, ssqk_row, feats_k, feats_skip, w_up, b_up,
              R=512):
    B, NU, _ = posu_pad.shape
    NK = posk_t.shape[2]
    CK = feats_k.shape[2]
    CS = feats_skip.shape[2]
    Cout = w_up.shape[0]
    R = min(R, NU)
    wa_t = w_up[:, :CK].T
    wb_t = w_up[:, CK:].T
    return pl.pallas_call(
        functools.partial(_up_body, R=R, NK=NK),
        grid=(B, NU // R),
        in_specs=[
            pl.BlockSpec((1, R, 8), lambda b, i: (b, i, 0)),
            pl.BlockSpec((1, 8, NK), lambda b, i: (b, 0, 0)),
            pl.BlockSpec((1, 1, NK), lambda b, i: (b, 0, 0)),
            pl.BlockSpec((1, NK, CK), lambda b, i: (b, 0, 0)),
            pl.BlockSpec((1, R, CS), lambda b, i: (b, i, 0)),
            pl.BlockSpec((CK, Cout), lambda b, i: (0, 0)),
            pl.BlockSpec((CS, Cout), lambda b, i: (0, 0)),
            pl.BlockSpec((1, Cout), lambda b, i: (0, 0)),
        ],
        out_specs=pl.BlockSpec((1, R, Cout), lambda b, i: (b, i, 0)),
        out_shape=jax.ShapeDtypeStruct((B, NU, Cout), jnp.float32),
    )(posu_pad, posk_t, ssqk_row, feats_k, feats_skip, wa_t, wb_t,
      b_up[None, :])


def _proj_kernel(f_ref, w_ref, b_ref, o_ref):
    o_ref[...] = jnp.dot(f_ref[...], w_ref[...],
                         preferred_element_type=jnp.float32) + b_ref[...]


def _out_proj(f, w_out, b_out):
    b, n, c = f.shape
    out = pl.pallas_call(
        _proj_kernel,
        out_shape=jax.ShapeDtypeStruct((b * n, w_out.shape[0]), jnp.float32),
    )(f.reshape(b * n, c), w_out.T, b_out[None, :])
    return out.reshape(b, n, w_out.shape[0])


def _bgather(feats, idx):
    return jax.vmap(lambda f, i: f[i])(feats, idx)


def _aug_of(x, pos3):
    ssq = jnp.sum(pos3 * pos3, axis=-1, keepdims=True)
    return jnp.concatenate([x, pos3, ssq], axis=-1), ssq


def _pads(pos3):
    B, N, _ = pos3.shape
    pad = jnp.concatenate(
        [pos3, jnp.zeros((B, N, 5), jnp.float32)], axis=-1)
    return pad, jnp.swapaxes(pad, 1, 2)


def kernel(x, pos, w_self0, w_edge0, w_self1, w_edge1, w_self2, w_edge2,
           w_up1, b_up1, w_self_u1, w_edge_u1, w_up0, b_up0, w_self_u0,
           w_edge_u0, w_out, b_out):
    B, N, _ = pos.shape
    pos0 = pos
    aug0, ssq0 = _aug_of(x, pos0)
    pad0, pt0 = _pads(pos0)
    ssq0r = jnp.swapaxes(ssq0, 1, 2)

    feat0, wm0 = _edge_conv(aug0, pad0, pt0, ssq0r, w_self0, w_edge0,
                            emit_mask=True)

    idx1, pos1_t, idx2, pos2_t = _fps_chain(jnp.swapaxes(pos0, 1, 2)[:, :3, :])
    pos1 = jnp.swapaxes(pos1_t, 1, 2)
    pos2 = jnp.swapaxes(pos2_t, 1, 2)

    feat0_g = _bgather(feat0, idx1)
    aug1, ssq1 = _aug_of(feat0_g, pos1)
    pad1, pt1 = _pads(pos1)
    ssq1r = jnp.swapaxes(ssq1, 1, 2)
    feat1, wm1 = _edge_conv(aug1, pad1, pt1, ssq1r, w_self1, w_edge1,
                            emit_mask=True)

    feat1_g = _bgather(feat1, idx2)
    aug2, ssq2 = _aug_of(feat1_g, pos2)
    pad2, pt2 = _pads(pos2)
    ssq2r = jnp.swapaxes(ssq2, 1, 2)
    feat2, _ = _edge_conv(aug2, pad2, pt2, ssq2r, w_self2, w_edge2)

    fu1 = _upsample(pad1, pt2, ssq2r, feat2, feat1, w_up1, b_up1)
    aug_u1, _ = _aug_of(fu1, pos1)
    fc1, _ = _edge_conv(aug_u1, pad1, pt1, ssq1r, w_self_u1, w_edge_u1,
                        wmask=wm1)

    fu0 = _upsample(pad0, pt1, ssq1r, fc1, feat0, w_up0, b_up0)
    aug_u0, _ = _aug_of(fu0, pos0)
    fc0, _ = _edge_conv(aug_u0, pad0, pt0, ssq0r, w_self_u0, w_edge_u0,
                        wmask=wm0)

    return _out_proj(fc0, w_out, b_out)


# speedup vs baseline: 1.6759x; 1.6759x over previous
import functools

import jax
import jax.numpy as jnp
from jax.experimental import pallas as pl

K = 16
_NEG_SLOPE = 0.2


def _leaky(v):
    return jnp.where(v >= 0, v, _NEG_SLOPE * v)


def _conv_body(posr_ref, pt_ref, ssqr_ref, a_ref, wst_ref, wet_ref,
               *refs, C, R, N, emit_mask, use_mask, k):
    if use_mask:
        wm_in_ref, out_ref = refs
    elif emit_mask:
        out_ref, wm_out_ref = refs
    else:
        (out_ref,) = refs
    i = pl.program_id(1)

    a_rows = a_ref[0, pl.ds(i * R, R), :]
    x_r = a_rows[:, :C]
    pos3_r = a_rows[:, C:C + 3]
    psq_r = a_rows[:, C + 3:C + 4]

    if use_mask:
        w = wm_in_ref[0]
    else:
        pos_r = posr_ref[0]
        ptm = pt_ref[0]
        pb = pos_r.astype(jnp.bfloat16).astype(jnp.float32)
        tb = ptm.astype(jnp.bfloat16).astype(jnp.float32)
        e = ((pb[:, 0:1] * tb[0:1, :] + pb[:, 1:2] * tb[1:2, :])
             + pb[:, 2:3] * tb[2:3, :])
        d = (psq_r + ssqr_ref[0]) - 2.0 * e
        d = jnp.maximum(d, 0.0)
        iota = jax.lax.broadcasted_iota(jnp.int32, (R, N), 1)
        mask = jnp.zeros((R, N), jnp.float32)
        for _ in range(k):
            dm = jnp.min(d, axis=1, keepdims=True)
            sel = jnp.min(jnp.where(d == dm, iota, N), axis=1, keepdims=True)
            hit = iota == sel
            mask = jnp.maximum(mask, hit.astype(jnp.float32))
            d = jnp.where(hit, jnp.inf, d)
        w = mask * jnp.float32(1.0 / k)
        if emit_mask:
            wm_out_ref[0] = w

    agg = jnp.dot(w, a_ref[0], preferred_element_type=jnp.float32)
    agg_x = agg[:, :C]
    agg_p = agg[:, C:C + 3]
    agg_sq = agg[:, C + 3:C + 4]
    rel_mean = pos3_r - agg_p
    meandsq = psq_r - 2.0 * jnp.sum(pos3_r * agg_p, axis=1,
                                    keepdims=True) + agg_sq
    g = jnp.concatenate([agg_x, rel_mean, meandsq], axis=1)
    val = (jnp.dot(x_r, wst_ref[...], preferred_element_type=jnp.float32)
           + jnp.dot(g, wet_ref[...], preferred_element_type=jnp.float32))
    out_ref[0] = _leaky(val)


def _edge_conv(aug, pos_pad, pos_t, ssq_row, w_self, w_edge, *,
               wmask=None, emit_mask=False, R=256, k=K):
    B, N, CA = aug.shape
    C = CA - 4
    Cout = w_self.shape[0]
    R = min(R, N)
    grid = (B, N // R)
    body = functools.partial(_conv_body, C=C, R=R, N=N,
                             emit_mask=emit_mask, use_mask=wmask is not None,
                             k=k)
    in_specs = [
        pl.BlockSpec((1, R, 8), lambda b, i: (b, i, 0)),
        pl.BlockSpec((1, 8, N), lambda b, i: (b, 0, 0)),
        pl.BlockSpec((1, 1, N), lambda b, i: (b, 0, 0)),
        pl.BlockSpec((1, N, CA), lambda b, i: (b, 0, 0)),
        pl.BlockSpec((C, Cout), lambda b, i: (0, 0)),
        pl.BlockSpec((CA, Cout), lambda b, i: (0, 0)),
    ]
    args = [pos_pad, pos_t, ssq_row, aug, w_self.T, w_edge.T]
    out_specs = [pl.BlockSpec((1, R, Cout), lambda b, i: (b, i, 0))]
    out_shapes = [jax.ShapeDtypeStruct((B, N, Cout), jnp.float32)]
    if wmask is not None:
        in_specs.append(pl.BlockSpec((1, R, N), lambda b, i: (b, i, 0)))
        args.append(wmask)
    elif emit_mask:
        out_specs.append(pl.BlockSpec((1, R, N), lambda b, i: (b, i, 0)))
        out_shapes.append(jax.ShapeDtypeStruct((B, N, N), jnp.float32))
    res = pl.pallas_call(
        body,
        grid=grid,
        in_specs=in_specs,
        out_specs=out_specs if len(out_specs) > 1 else out_specs[0],
        out_shape=out_shapes if len(out_shapes) > 1 else out_shapes[0],
    )(*args)
    if emit_mask and wmask is None:
        return res[0], res[1]
    return (res[0] if isinstance(res, (list, tuple)) else res), None


def _red2(a, op):
    r = op(a, axis=2, keepdims=True)
    return op(r, axis=1, keepdims=True)


def _fps_body(pos_ref, idx1_ref, sel1_ref, idx2_ref, sel2_ref, *, N, N1, N2):
    B = pos_ref.shape[0]
    xs = pos_ref[:, 0]
    ys = pos_ref[:, 1]
    zs = pos_ref[:, 2]

    def lin_of(n):
        m = n // 8
        return (jax.lax.broadcasted_iota(jnp.int32, (B, 8, m), 1) * m
                + jax.lax.broadcasted_iota(jnp.int32, (B, 8, m), 2))

    def run(xc, yc, zc, npoint, n):
        lin = lin_of(n)
        lin_sel = lin_of(npoint)

        def body(t, state):
            dists, idxacc, sx, sy, sz, lx, ly, lz = state
            dx = xc - lx
            dy = yc - ly
            dz = zc - lz
            d = (dx * dx + dy * dy) + dz * dz
            dists = jnp.minimum(dists, d)
            maxv = _red2(dists, jnp.max)
            nxt = _red2(jnp.where(dists == maxv, lin, n), jnp.min)
            hit = lin == nxt
            lx = _red2(jnp.where(hit, xc, 0.0), jnp.sum)
            ly = _red2(jnp.where(hit, yc, 0.0), jnp.sum)
            lz = _red2(jnp.where(hit, zc, 0.0), jnp.sum)
            at = lin_sel == t
            idxacc = jnp.where(at, nxt, idxacc)
            sx = jnp.where(at, lx, sx)
            sy = jnp.where(at, ly, sy)
            sz = jnp.where(at, lz, sz)
            return dists, idxacc, sx, sy, sz, lx, ly, lz

        lx0 = xc[:, 0:1, 0:1]
        ly0 = yc[:, 0:1, 0:1]
        lz0 = zc[:, 0:1, 0:1]
        m_sel = npoint // 8
        at0 = lin_sel == 0
        z_i = jnp.zeros((B, 8, m_sel), jnp.int32)
        sx0 = jnp.where(at0, lx0, jnp.zeros((B, 8, m_sel), jnp.float32))
        sy0 = jnp.where(at0, ly0, jnp.zeros((B, 8, m_sel), jnp.float32))
        sz0 = jnp.where(at0, lz0, jnp.zeros((B, 8, m_sel), jnp.float32))
        state = (jnp.full((B, 8, n // 8), 1e10, jnp.float32), z_i,
                 sx0, sy0, sz0, lx0, ly0, lz0)
        state = jax.lax.fori_loop(1, npoint, body, state)
        return state[1], state[2], state[3], state[4]

    idx1, s1x, s1y, s1z = run(xs, ys, zs, N1, N)
    idx1_ref[...] = idx1
    sel1_ref[:, 0] = s1x
    sel1_ref[:, 1] = s1y
    sel1_ref[:, 2] = s1z
    idx2, s2x, s2y, s2z = run(s1x, s1y, s1z, N2, N1)
    idx2_ref[...] = idx2
    sel2_ref[:, 0] = s2x
    sel2_ref[:, 1] = s2y
    sel2_ref[:, 2] = s2z


def _fps_chain(pos_t):
    B, _, N = pos_t.shape
    N1, N2 = N // 2, N // 4
    pos4 = pos_t.reshape(B, 3, 8, N // 8)
    idx1, sel1, idx2, sel2 = pl.pallas_call(
        functools.partial(_fps_body, N=N, N1=N1, N2=N2),
        out_shape=(
            jax.ShapeDtypeStruct((B, 8, N1 // 8), jnp.int32),
            jax.ShapeDtypeStruct((B, 3, 8, N1 // 8), jnp.float32),
            jax.ShapeDtypeStruct((B, 8, N2 // 8), jnp.int32),
            jax.ShapeDtypeStruct((B, 3, 8, N2 // 8), jnp.float32),
        ),
    )(pos4)
    return (idx1.reshape(B, N1), sel1.reshape(B, 3, N1),
            idx2.reshape(B, N2), sel2.reshape(B, 3, N2))


def _up_body(posr_ref, pt_ref, ssqr_ref, fk_ref, fs_ref, wat_ref, wbt_ref,
             b_ref, out_ref, *, R, NK):
    pos_r = posr_ref[0]
    ptm = pt_ref[0]
    psq_r = ((pos_r[:, 0:1] * pos_r[:, 0:1] + pos_r[:, 1:2] * pos_r[:, 1:2])
             + pos_r[:, 2:3] * pos_r[:, 2:3])
    pb = pos_r.astype(jnp.bfloat16).astype(jnp.float32)
    tb = ptm.astype(jnp.bfloat16).astype(jnp.float32)
    e = ((pb[:, 0:1] * tb[0:1, :] + pb[:, 1:2] * tb[1:2, :])
         + pb[:, 2:3] * tb[2:3, :])
    d = (psq_r + ssqr_ref[0]) - 2.0 * e
    d = jnp.maximum(d, 0.0)
    dd = d
    iota = jax.lax.broadcasted_iota(jnp.int32, (R, NK), 1)
    mask = jnp.zeros((R, NK), jnp.float32)
    for _ in range(3):
        dm = jnp.min(dd, axis=1, keepdims=True)
        sel = jnp.min(jnp.where(dd == dm, iota, NK), axis=1, keepdims=True)
        hit = iota == sel
        mask = jnp.maximum(mask, hit.astype(jnp.float32))
        dd = jnp.where(hit, jnp.inf, dd)
    wfull = mask / (d + 1e-8)
    w3 = wfull / jnp.sum(wfull, axis=1, keepdims=True)
    interp = jnp.dot(w3, fk_ref[0], preferred_element_type=jnp.float32)
    val = (jnp.dot(interp, wat_ref[...], preferred_element_type=jnp.float32)
           + jnp.dot(fs_ref[0], wbt_ref[...],
                     preferred_element_type=jnp.float32)
           + b_ref[...])
    out_ref[0] = _leaky(val)


def _upsample(posu_pad, posk_t, ssqk_row, feats_k, feats_skip, w_up, b_up,
              R=512):
    B, NU, _ = posu_pad.shape
    NK = posk_t.shape[2]
    CK = feats_k.shape[2]
    CS = feats_skip.shape[2]
    Cout = w_up.shape[0]
    R = min(R, NU)
    wa_t = w_up[:, :CK].T
    wb_t = w_up[:, CK:].T
    return pl.pallas_call(
        functools.partial(_up_body, R=R, NK=NK),
        grid=(B, NU // R),
        in_specs=[
            pl.BlockSpec((1, R, 8), lambda b, i: (b, i, 0)),
            pl.BlockSpec((1, 8, NK), lambda b, i: (b, 0, 0)),
            pl.BlockSpec((1, 1, NK), lambda b, i: (b, 0, 0)),
            pl.BlockSpec((1, NK, CK), lambda b, i: (b, 0, 0)),
            pl.BlockSpec((1, R, CS), lambda b, i: (b, i, 0)),
            pl.BlockSpec((CK, Cout), lambda b, i: (0, 0)),
            pl.BlockSpec((CS, Cout), lambda b, i: (0, 0)),
            pl.BlockSpec((1, Cout), lambda b, i: (0, 0)),
        ],
        out_specs=pl.BlockSpec((1, R, Cout), lambda b, i: (b, i, 0)),
        out_shape=jax.ShapeDtypeStruct((B, NU, Cout), jnp.float32),
    )(posu_pad, posk_t, ssqk_row, feats_k, feats_skip, wa_t, wb_t,
      b_up[None, :])


def _proj_kernel(f_ref, w_ref, b_ref, o_ref):
    o_ref[...] = jnp.dot(f_ref[...], w_ref[...],
                         preferred_element_type=jnp.float32) + b_ref[...]


def _out_proj(f, w_out, b_out):
    b, n, c = f.shape
    out = pl.pallas_call(
        _proj_kernel,
        out_shape=jax.ShapeDtypeStruct((b * n, w_out.shape[0]), jnp.float32),
    )(f.reshape(b * n, c), w_out.T, b_out[None, :])
    return out.reshape(b, n, w_out.shape[0])


def _bgather(feats, idx):
    return jax.vmap(lambda f, i: f[i])(feats, idx)


def _aug_of(x, pos3):
    ssq = jnp.sum(pos3 * pos3, axis=-1, keepdims=True)
    return jnp.concatenate([x, pos3, ssq], axis=-1), ssq


def _pads(pos3):
    B, N, _ = pos3.shape
    pad = jnp.concatenate(
        [pos3, jnp.zeros((B, N, 5), jnp.float32)], axis=-1)
    return pad, jnp.swapaxes(pad, 1, 2)


def kernel(x, pos, w_self0, w_edge0, w_self1, w_edge1, w_self2, w_edge2,
           w_up1, b_up1, w_self_u1, w_edge_u1, w_up0, b_up0, w_self_u0,
           w_edge_u0, w_out, b_out):
    B, N, _ = pos.shape
    pos0 = pos
    aug0, ssq0 = _aug_of(x, pos0)
    pad0, pt0 = _pads(pos0)
    ssq0r = jnp.swapaxes(ssq0, 1, 2)

    feat0, wm0 = _edge_conv(aug0, pad0, pt0, ssq0r, w_self0, w_edge0,
                            emit_mask=True)

    idx1, pos1_t, idx2, pos2_t = _fps_chain(jnp.swapaxes(pos0, 1, 2)[:, :3, :])
    pos1 = jnp.swapaxes(pos1_t, 1, 2)
    pos2 = jnp.swapaxes(pos2_t, 1, 2)

    feat0_g = _bgather(feat0, idx1)
    aug1, ssq1 = _aug_of(feat0_g, pos1)
    pad1, pt1 = _pads(pos1)
    ssq1r = jnp.swapaxes(ssq1, 1, 2)
    feat1, wm1 = _edge_conv(aug1, pad1, pt1, ssq1r, w_self1, w_edge1,
                            emit_mask=True)

    feat1_g = _bgather(feat1, idx2)
    aug2, ssq2 = _aug_of(feat1_g, pos2)
    pad2, pt2 = _pads(pos2)
    ssq2r = jnp.swapaxes(ssq2, 1, 2)
    feat2, _ = _edge_conv(aug2, pad2, pt2, ssq2r, w_self2, w_edge2)

    fu1 = _upsample(pad1, pt2, ssq2r, feat2, feat1, w_up1, b_up1)
    aug_u1, _ = _aug_of(fu1, pos1)
    fc1, _ = _edge_conv(aug_u1, pad1, pt1, ssq1r, w_self_u1, w_edge_u1,
                        wmask=wm1)

    fu0 = _upsample(pad0, pt1, ssq1r, fc1, feat0, w_up0, b_up0)
    aug_u0, _ = _aug_of(fu0, pos0)
    fc0, _ = _edge_conv(aug_u0, pad0, pt0, ssq0r, w_self_u0, w_edge_u0,
                        wmask=wm0)

    return _out_proj(fc0, w_out, b_out)


# speedup vs baseline: 1.6852x; 1.0056x over previous
import functools

import jax
import jax.numpy as jnp
from jax.experimental import pallas as pl

K = 16
_NEG_SLOPE = 0.2


def _leaky(v):
    return jnp.where(v >= 0, v, _NEG_SLOPE * v)


def _conv_body(posr_ref, pt_ref, ssqr_ref, a_ref, wst_ref, wet_ref,
               *refs, C, R, N, emit_mask, use_mask, k):
    if use_mask:
        wm_in_ref, out_ref = refs
    elif emit_mask:
        out_ref, wm_out_ref = refs
    else:
        (out_ref,) = refs
    i = pl.program_id(1)

    a_rows = a_ref[0, pl.ds(i * R, R), :]
    x_r = a_rows[:, :C]
    pos3_r = a_rows[:, C:C + 3]
    psq_r = a_rows[:, C + 3:C + 4]

    if use_mask:
        w = wm_in_ref[0].astype(jnp.float32)
    else:
        pos_r = posr_ref[0]
        ptm = pt_ref[0]
        pb = pos_r.astype(jnp.bfloat16).astype(jnp.float32)
        tb = ptm.astype(jnp.bfloat16).astype(jnp.float32)
        e = ((pb[:, 0:1] * tb[0:1, :] + pb[:, 1:2] * tb[1:2, :])
             + pb[:, 2:3] * tb[2:3, :])
        d = (psq_r + ssqr_ref[0]) - 2.0 * e
        d = jnp.maximum(d, 0.0)
        iota = jax.lax.broadcasted_iota(jnp.int32, (R, N), 1)
        mask = jnp.zeros((R, N), jnp.float32)
        for _ in range(k):
            dm = jnp.min(d, axis=1, keepdims=True)
            sel = jnp.min(jnp.where(d == dm, iota, N), axis=1, keepdims=True)
            hit = iota == sel
            mask = jnp.maximum(mask, hit.astype(jnp.float32))
            d = jnp.where(hit, jnp.inf, d)
        w = mask * jnp.float32(1.0 / k)
        if emit_mask:
            wm_out_ref[0] = w.astype(jnp.bfloat16)

    agg = jnp.dot(w, a_ref[0], preferred_element_type=jnp.float32)
    agg_x = agg[:, :C]
    agg_p = agg[:, C:C + 3]
    agg_sq = agg[:, C + 3:C + 4]
    rel_mean = pos3_r - agg_p
    meandsq = psq_r - 2.0 * jnp.sum(pos3_r * agg_p, axis=1,
                                    keepdims=True) + agg_sq
    g = jnp.concatenate([agg_x, rel_mean, meandsq], axis=1)
    val = (jnp.dot(x_r, wst_ref[...], preferred_element_type=jnp.float32)
           + jnp.dot(g, wet_ref[...], preferred_element_type=jnp.float32))
    out_ref[0] = _leaky(val)


def _edge_conv(aug, pos_pad, pos_t, ssq_row, w_self, w_edge, *,
               wmask=None, emit_mask=False, R=256, k=K):
    B, N, CA = aug.shape
    C = CA - 4
    Cout = w_self.shape[0]
    R = min(R, N)
    grid = (B, N // R)
    body = functools.partial(_conv_body, C=C, R=R, N=N,
                             emit_mask=emit_mask, use_mask=wmask is not None,
                             k=k)
    in_specs = [
        pl.BlockSpec((1, R, 8), lambda b, i: (b, i, 0)),
        pl.BlockSpec((1, 8, N), lambda b, i: (b, 0, 0)),
        pl.BlockSpec((1, 1, N), lambda b, i: (b, 0, 0)),
        pl.BlockSpec((1, N, CA), lambda b, i: (b, 0, 0)),
        pl.BlockSpec((C, Cout), lambda b, i: (0, 0)),
        pl.BlockSpec((CA, Cout), lambda b, i: (0, 0)),
    ]
    args = [pos_pad, pos_t, ssq_row, aug, w_self.T, w_edge.T]
    out_specs = [pl.BlockSpec((1, R, Cout), lambda b, i: (b, i, 0))]
    out_shapes = [jax.ShapeDtypeStruct((B, N, Cout), jnp.float32)]
    if wmask is not None:
        in_specs.append(pl.BlockSpec((1, R, N), lambda b, i: (b, i, 0)))
        args.append(wmask)
    elif emit_mask:
        out_specs.append(pl.BlockSpec((1, R, N), lambda b, i: (b, i, 0)))
        out_shapes.append(jax.ShapeDtypeStruct((B, N, N), jnp.bfloat16))
    res = pl.pallas_call(
        body,
        grid=grid,
        in_specs=in_specs,
        out_specs=out_specs if len(out_specs) > 1 else out_specs[0],
        out_shape=out_shapes if len(out_shapes) > 1 else out_shapes[0],
    )(*args)
    if emit_mask and wmask is None:
        return res[0], res[1]
    return (res[0] if isinstance(res, (list, tuple)) else res), None


def _red2(a, op):
    r = op(a, axis=2, keepdims=True)
    return op(r, axis=1, keepdims=True)


def _fps_body(pos_ref, idx1_ref, sel1_ref, idx2_ref, sel2_ref, *, N, N1, N2):
    B = pos_ref.shape[0]
    xs = pos_ref[:, 0]
    ys = pos_ref[:, 1]
    zs = pos_ref[:, 2]

    def lin_of(n):
        m = n // 8
        return (jax.lax.broadcasted_iota(jnp.int32, (B, 8, m), 1) * m
                + jax.lax.broadcasted_iota(jnp.int32, (B, 8, m), 2))

    def run(xc, yc, zc, npoint, n):
        lin = lin_of(n)
        lin_sel = lin_of(npoint)

        def body(t, state):
            dists, idxacc, sx, sy, sz, lx, ly, lz = state
            dx = xc - lx
            dy = yc - ly
            dz = zc - lz
            d = (dx * dx + dy * dy) + dz * dz
            dists = jnp.minimum(dists, d)
            maxv = _red2(dists, jnp.max)
            nxt = _red2(jnp.where(dists == maxv, lin, n), jnp.min)
            hit = lin == nxt
            lx = _red2(jnp.where(hit, xc, 0.0), jnp.sum)
            ly = _red2(jnp.where(hit, yc, 0.0), jnp.sum)
            lz = _red2(jnp.where(hit, zc, 0.0), jnp.sum)
            at = lin_sel == t
            idxacc = jnp.where(at, nxt, idxacc)
            sx = jnp.where(at, lx, sx)
            sy = jnp.where(at, ly, sy)
            sz = jnp.where(at, lz, sz)
            return dists, idxacc, sx, sy, sz, lx, ly, lz

        lx0 = xc[:, 0:1, 0:1]
        ly0 = yc[:, 0:1, 0:1]
        lz0 = zc[:, 0:1, 0:1]
        m_sel = npoint // 8
        at0 = lin_sel == 0
        z_i = jnp.zeros((B, 8, m_sel), jnp.int32)
        sx0 = jnp.where(at0, lx0, jnp.zeros((B, 8, m_sel), jnp.float32))
        sy0 = jnp.where(at0, ly0, jnp.zeros((B, 8, m_sel), jnp.float32))
        sz0 = jnp.where(at0, lz0, jnp.zeros((B, 8, m_sel), jnp.float32))
        state = (jnp.full((B, 8, n // 8), 1e10, jnp.float32), z_i,
                 sx0, sy0, sz0, lx0, ly0, lz0)
        state = jax.lax.fori_loop(1, npoint, body, state)
        return state[1], state[2], state[3], state[4]

    idx1, s1x, s1y, s1z = run(xs, ys, zs, N1, N)
    idx1_ref[...] = idx1
    sel1_ref[:, 0] = s1x
    sel1_ref[:, 1] = s1y
    sel1_ref[:, 2] = s1z
    idx2, s2x, s2y, s2z = run(s1x, s1y, s1z, N2, N1)
    idx2_ref[...] = idx2
    sel2_ref[:, 0] = s2x
    sel2_ref[:, 1] = s2y
    sel2_ref[:, 2] = s2z


def _fps_chain(pos_t):
    B, _, N = pos_t.shape
    N1, N2 = N // 2, N // 4
    pos4 = pos_t.reshape(B, 3, 8, N // 8)
    idx1, sel1, idx2, sel2 = pl.pallas_call(
        functools.partial(_fps_body, N=N, N1=N1, N2=N2),
        out_shape=(
            jax.ShapeDtypeStruct((B, 8, N1 // 8), jnp.int32),
            jax.ShapeDtypeStruct((B, 3, 8, N1 // 8), jnp.float32),
            jax.ShapeDtypeStruct((B, 8, N2 // 8), jnp.int32),
            jax.ShapeDtypeStruct((B, 3, 8, N2 // 8), jnp.float32),
        ),
    )(pos4)
    return (idx1.reshape(B, N1), sel1.reshape(B, 3, N1),
            idx2.reshape(B, N2), sel2.reshape(B, 3, N2))


def _up_body(posr_ref, pt_ref, ssqr_ref, fk_ref, fs_ref, wat_ref, wbt_ref,
             b_ref, out_ref, *, R, NK):
    pos_r = posr_ref[0]
    ptm = pt_ref[0]
    psq_r = ((pos_r[:, 0:1] * pos_r[:, 0:1] + pos_r[:, 1:2] * pos_r[:, 1:2])
             + pos_r[:, 2:3] * pos_r[:, 2:3])
    pb = pos_r.astype(jnp.bfloat16).astype(jnp.float32)
    tb = ptm.astype(jnp.bfloat16).astype(jnp.float32)
    e = ((pb[:, 0:1] * tb[0:1, :] + pb[:, 1:2] * tb[1:2, :])
         + pb[:, 2:3] * tb[2:3, :])
    d = (psq_r + ssqr_ref[0]) - 2.0 * e
    d = jnp.maximum(d, 0.0)
    dd = d
    iota = jax.lax.broadcasted_iota(jnp.int32, (R, NK), 1)
    mask = jnp.zeros((R, NK), jnp.float32)
    for _ in range(3):
        dm = jnp.min(dd, axis=1, keepdims=True)
        sel = jnp.min(jnp.where(dd == dm, iota, NK), axis=1, keepdims=True)
        hit = iota == sel
        mask = jnp.maximum(mask, hit.astype(jnp.float32))
        dd = jnp.where(hit, jnp.inf, dd)
    wfull = mask / (d + 1e-8)
    w3 = wfull / jnp.sum(wfull, axis=1, keepdims=True)
    interp = jnp.dot(w3, fk_ref[0], preferred_element_type=jnp.float32)
    val = (jnp.dot(interp, wat_ref[...], preferred_element_type=jnp.float32)
           + jnp.dot(fs_ref[0], wbt_ref[...],
                     preferred_element_type=jnp.float32)
           + b_ref[...])
    out_ref[0] = _leaky(val)


def _upsample(posu_pad, posk_t, ssqk_row, feats_k, feats_skip, w_up, b_up,
              R=512):
    B, NU, _ = posu_pad.shape
    NK = posk_t.shape[2]
    CK = feats_k.shape[2]
    CS = feats_skip.shape[2]
    Cout = w_up.shape[0]
    R = min(R, NU)
    wa_t = w_up[:, :CK].T
    wb_t = w_up[:, CK:].T
    return pl.pallas_call(
        functools.partial(_up_body, R=R, NK=NK),
        grid=(B, NU // R),
        in_specs=[
            pl.BlockSpec((1, R, 8), lambda b, i: (b, i, 0)),
            pl.BlockSpec((1, 8, NK), lambda b, i: (b, 0, 0)),
            pl.BlockSpec((1, 1, NK), lambda b, i: (b, 0, 0)),
            pl.BlockSpec((1, NK, CK), lambda b, i: (b, 0, 0)),
            pl.BlockSpec((1, R, CS), lambda b, i: (b, i, 0)),
            pl.BlockSpec((CK, Cout), lambda b, i: (0, 0)),
            pl.BlockSpec((CS, Cout), lambda b, i: (0, 0)),
            pl.BlockSpec((1, Cout), lambda b, i: (0, 0)),
        ],
        out_specs=pl.BlockSpec((1, R, Cout), lambda b, i: (b, i, 0)),
        out_shape=jax.ShapeDtypeStruct((B, NU, Cout), jnp.float32),
    )(posu_pad, posk_t, ssqk_row, feats_k, feats_skip, wa_t, wb_t,
      b_up[None, :])


def _proj_kernel(f_ref, w_ref, b_ref, o_ref):
    o_ref[...] = jnp.dot(f_ref[...], w_ref[...],
                         preferred_element_type=jnp.float32) + b_ref[...]


def _out_proj(f, w_out, b_out):
    b, n, c = f.shape
    out = pl.pallas_call(
        _proj_kernel,
        out_shape=jax.ShapeDtypeStruct((b * n, w_out.shape[0]), jnp.float32),
    )(f.reshape(b * n, c), w_out.T, b_out[None, :])
    return out.reshape(b, n, w_out.shape[0])


def _bgather(feats, idx):
    return jax.vmap(lambda f, i: f[i])(feats, idx)


def _aug_of(x, pos3):
    ssq = jnp.sum(pos3 * pos3, axis=-1, keepdims=True)
    return jnp.concatenate([x, pos3, ssq], axis=-1), ssq


def _pads(pos3):
    B, N, _ = pos3.shape
    pad = jnp.concatenate(
        [pos3, jnp.zeros((B, N, 5), jnp.float32)], axis=-1)
    return pad, jnp.swapaxes(pad, 1, 2)


def kernel(x, pos, w_self0, w_edge0, w_self1, w_edge1, w_self2, w_edge2,
           w_up1, b_up1, w_self_u1, w_edge_u1, w_up0, b_up0, w_self_u0,
           w_edge_u0, w_out, b_out):
    B, N, _ = pos.shape
    pos0 = pos
    aug0, ssq0 = _aug_of(x, pos0)
    pad0, pt0 = _pads(pos0)
    ssq0r = jnp.swapaxes(ssq0, 1, 2)

    feat0, wm0 = _edge_conv(aug0, pad0, pt0, ssq0r, w_self0, w_edge0,
                            emit_mask=True)

    idx1, pos1_t, idx2, pos2_t = _fps_chain(jnp.swapaxes(pos0, 1, 2)[:, :3, :])
    pos1 = jnp.swapaxes(pos1_t, 1, 2)
    pos2 = jnp.swapaxes(pos2_t, 1, 2)

    feat0_g = _bgather(feat0, idx1)
    aug1, ssq1 = _aug_of(feat0_g, pos1)
    pad1, pt1 = _pads(pos1)
    ssq1r = jnp.swapaxes(ssq1, 1, 2)
    feat1, wm1 = _edge_conv(aug1, pad1, pt1, ssq1r, w_self1, w_edge1,
                            emit_mask=True)

    feat1_g = _bgather(feat1, idx2)
    aug2, ssq2 = _aug_of(feat1_g, pos2)
    pad2, pt2 = _pads(pos2)
    ssq2r = jnp.swapaxes(ssq2, 1, 2)
    feat2, _ = _edge_conv(aug2, pad2, pt2, ssq2r, w_self2, w_edge2)

    fu1 = _upsample(pad1, pt2, ssq2r, feat2, feat1, w_up1, b_up1)
    aug_u1, _ = _aug_of(fu1, pos1)
    fc1, _ = _edge_conv(aug_u1, pad1, pt1, ssq1r, w_self_u1, w_edge_u1,
                        wmask=wm1)

    fu0 = _upsample(pad0, pt1, ssq1r, fc1, feat0, w_up0, b_up0)
    aug_u0, _ = _aug_of(fu0, pos0)
    fc0, _ = _edge_conv(aug_u0, pad0, pt0, ssq0r, w_self_u0, w_edge_u0,
                        wmask=wm0)

    return _out_proj(fc0, w_out, b_out)
